# baseline (speedup 1.0000x reference)
import functools

import jax
import jax.numpy as jnp
from jax import lax
from jax.experimental import pallas as pl
from jax.experimental.pallas import tpu as pltpu
from jax.experimental.pallas import tpu_sc as plsc

NC = 2
NS = 16
NW = NC * NS
NPAD = 10240
K = 128
F = 128


def _sc_mesh():
    return plsc.VectorSubcoreMesh(
        core_axis_name="c", subcore_axis_name="s", num_cores=NC, num_subcores=NS
    )


def _make_deg_kernel(n_edges):
    epw = n_edges // NW
    nchunks = epw // K
    assert nchunks % 4 == 0 and epw % K == 0
    ept = NPAD // NS

    @functools.partial(
        pl.kernel,
        out_type=jax.ShapeDtypeStruct((NC, NPAD), jnp.float32),
        mesh=_sc_mesh(),
        scratch_types=[
            [pltpu.VMEM((K,), jnp.int32) for _ in range(4)],
            pltpu.VMEM((K,), jnp.float32),
            pltpu.VMEM((ept,), jnp.float32),
            pltpu.VMEM_SHARED((NPAD,), jnp.float32),
            [pltpu.SemaphoreType.DMA for _ in range(4)],
            [pltpu.SemaphoreType.DMA for _ in range(4)],
        ],
    )
    def deg_kernel(dst_hbm, out_hbm, dstb, ones, zb, acc, semD, semS):
        c = lax.axis_index("c")
        s = lax.axis_index("s")
        w = c * NS + s
        base = w * epw
        for q in range(4):
            pltpu.async_copy(dst_hbm.at[pl.ds(base + q * K, K)], dstb[q], semD[q])
        one16 = jnp.full((16,), 1.0, jnp.float32)
        zero16 = jnp.zeros((16,), jnp.float32)
        for j in range(K // 16):
            ones[pl.ds(j * 16, 16)] = one16

        def zfill(i, carry):
            zb[pl.ds(i * 16, 16)] = zero16
            return carry

        lax.fori_loop(0, ept // 16, zfill, 0)
        pltpu.sync_copy(zb, acc.at[pl.ds(s * ept, ept)])
        plsc.subcore_barrier()

        def body(i, carry):
            c0 = 4 * i
            for q in range(4):
                pltpu.make_async_copy(
                    dst_hbm.at[pl.ds(base + (c0 + q) * K, K)], dstb[q], semD[q]
                ).wait()
                pltpu.async_copy(ones, acc.at[dstb[q]], semS[q], add=True)
            for q in range(4):
                pltpu.make_async_copy(ones, acc.at[dstb[q]], semS[q]).wait()

                @pl.when(c0 + q + 4 < nchunks)
                def _():
                    pltpu.async_copy(
                        dst_hbm.at[pl.ds(base + (c0 + q + 4) * K, K)],
                        dstb[q], semD[q],
                    )

            return carry

        lax.fori_loop(0, nchunks // 4, body, 0)
        plsc.subcore_barrier()
        pltpu.sync_copy(acc.at[pl.ds(s * ept, ept)], zb)
        pltpu.sync_copy(zb, out_hbm.at[c].at[pl.ds(s * ept, ept)])

    return deg_kernel


KS = 64
RD = 5
GD = RD // 2
IR = 2 * RD


def _make_scatter_kernel(n_edges):
    epw = n_edges // NW
    nchunks = epw // KS
    assert nchunks % IR == 0 and epw % KS == 0
    rpt = NPAD // NS
    bounce_rows = 40

    @functools.partial(
        pl.kernel,
        out_type=jax.ShapeDtypeStruct((NC, NPAD, F), jnp.float32),
        mesh=_sc_mesh(),
        scratch_types=[
            [pltpu.VMEM((KS,), jnp.int32) for _ in range(IR)],
            [pltpu.VMEM((KS,), jnp.int32) for _ in range(IR)],
            [pltpu.VMEM((KS, F), jnp.float32) for _ in range(RD)],
            pltpu.VMEM((bounce_rows, F), jnp.float32),
            pltpu.VMEM_SHARED((NPAD, F), jnp.float32),
            [pltpu.SemaphoreType.DMA for _ in range(IR)],
            [pltpu.SemaphoreType.DMA for _ in range(RD)],
            [pltpu.SemaphoreType.DMA for _ in range(RD)],
            pltpu.SemaphoreType.DMA,
        ],
    )
    def scatter_kernel(src_hbm, dst_hbm, u_hbm, out_hbm,
                       srcb, dstb, rows, bounce, acc,
                       semI, semG, semS, semZ):
        c = lax.axis_index("c")
        s = lax.axis_index("s")
        w = c * NS + s
        base = w * epw

        def start_idx(ch, slot):
            pltpu.async_copy(src_hbm.at[pl.ds(base + ch * KS, KS)],
                             srcb[slot], semI[slot])
            pltpu.async_copy(dst_hbm.at[pl.ds(base + ch * KS, KS)],
                             dstb[slot], semI[slot])

        def wait_idx(ch, slot):
            pltpu.make_async_copy(src_hbm.at[pl.ds(base + ch * KS, KS)],
                                  srcb[slot], semI[slot]).wait()
            pltpu.make_async_copy(dst_hbm.at[pl.ds(base + ch * KS, KS)],
                                  dstb[slot], semI[slot]).wait()

        def start_gather(slot8, slot4):
            pltpu.async_copy(u_hbm.at[srcb[slot8]], rows[slot4], semG[slot4])

        def wait_gather(slot8, slot4):
            pltpu.make_async_copy(u_hbm.at[srcb[slot8]], rows[slot4],
                                  semG[slot4]).wait()

        def start_scatter(slot8, slot4):
            pltpu.async_copy(rows[slot4], acc.at[dstb[slot8]], semS[slot4],
                             add=True)

        def wait_scatter(slot8, slot4):
            pltpu.make_async_copy(rows[slot4], acc.at[dstb[slot8]],
                                  semS[slot4]).wait()

        for q in range(IR - GD):
            start_idx(q, q)
        for q in range(GD):
            wait_idx(q, q)
            start_gather(q, q)

        zero16 = jnp.zeros((16,), jnp.float32)

        def zfill(i, carry):
            for j in range(F // 16):
                bounce[i, pl.ds(j * 16, 16)] = zero16
            return carry

        lax.fori_loop(0, bounce_rows, zfill, 0)
        for t in range(rpt // bounce_rows):
            pltpu.async_copy(
                bounce, acc.at[pl.ds(s * rpt + t * bounce_rows, bounce_rows)],
                semZ,
            )
        for t in range(rpt // bounce_rows):
            pltpu.make_async_copy(
                bounce, acc.at[pl.ds(s * rpt + t * bounce_rows, bounce_rows)],
                semZ,
            ).wait()
        plsc.subcore_barrier()

        def body(i, carry):
            c0 = IR * i
            for q in range(IR):
                ch = c0 + q

                @pl.when(ch >= GD)
                def _():
                    wait_scatter((q - GD) % IR, (q - GD) % RD)

                @pl.when(ch + IR - GD < nchunks)
                def _():
                    start_idx(ch + IR - GD, (q - GD) % IR)

                @pl.when(ch + GD < nchunks)
                def _():
                    wait_idx(ch + GD, (q + GD) % IR)
                    start_gather((q + GD) % IR, (q + GD) % RD)

                wait_gather(q, q % RD)
                start_scatter(q, q % RD)
            return carry

        lax.fori_loop(0, nchunks // IR, body, 0)
        for ch in range(nchunks - GD, nchunks):
            wait_scatter(ch % IR, ch % RD)
        plsc.subcore_barrier()
        pltpu.async_copy(acc.at[pl.ds(s * rpt, rpt)],
                         out_hbm.at[c].at[pl.ds(s * rpt, rpt)], semZ)
        pltpu.make_async_copy(acc.at[pl.ds(s * rpt, rpt)],
                              out_hbm.at[c].at[pl.ds(s * rpt, rpt)], semZ).wait()

    return scatter_kernel


def _tc_first(x, degp_t):
    n = x.shape[0]

    def body(x_ref, degp_ref, dinv_ref, u_ref):
        p = degp_ref[...]
        deg = p[:, 0:1] + p[:, 1:2] + 1.0
        dinv = lax.rsqrt(deg)[:n]
        d2 = jnp.broadcast_to(dinv, (n, F))
        dinv_ref[...] = d2
        u_ref[...] = d2 * x_ref[...]

    return pl.pallas_call(
        body,
        out_shape=(
            jax.ShapeDtypeStruct((n, F), jnp.float32),
            jax.ShapeDtypeStruct((n, F), jnp.float32),
        ),
    )(x, degp_t)


def _tc_layer(S, u, dinv2d, W, b, g, be):
    n = u.shape[0]
    fout = W.shape[1]

    def body(s_ref, u_ref, dinv_ref, w_ref, b_ref, g_ref, be_ref, out_ref):
        d2 = dinv_ref[...]
        agg = s_ref[0, :n, :] + s_ref[1, :n, :] + u_ref[...]
        h = jnp.dot(agg, w_ref[...], preferred_element_type=jnp.float32)
        t = d2[:, :fout] * h + b_ref[...]
        t = jnp.maximum(t, 0.0)
        mu = jnp.mean(t, axis=0, keepdims=True)
        var = jnp.mean((t - mu) ** 2, axis=0, keepdims=True)
        t = g_ref[...] * (t - mu) * lax.rsqrt(var + 1e-5) + be_ref[...]
        out_ref[:, :fout] = d2[:, :fout] * t
        if fout < F:
            out_ref[:, fout:] = jnp.zeros((n, F - fout), jnp.float32)

    return pl.pallas_call(
        body,
        out_shape=jax.ShapeDtypeStruct((n, F), jnp.float32),
    )(S, u, dinv2d, W, b.reshape(1, fout), g.reshape(1, fout), be.reshape(1, fout))


def _tc_final(S, u, dinv2d, W4p, b4):
    n = u.shape[0]

    def body(s_ref, u_ref, dinv_ref, w_ref, b_ref, out_ref):
        d2 = dinv_ref[...]
        agg = s_ref[0, :n, :] + s_ref[1, :n, :] + u_ref[...]
        h = jnp.dot(agg, w_ref[...], preferred_element_type=jnp.float32)
        t = d2[:, :2] * h + b_ref[...]
        a = t[:, 0:1]
        bb = t[:, 1:2]
        m = jnp.maximum(a, bb)
        lse = m + jnp.log(jnp.exp(a - m) + jnp.exp(bb - m))
        out_ref[...] = jnp.concatenate([a - lse, bb - lse], axis=1)

    return pl.pallas_call(
        body,
        out_shape=jax.ShapeDtypeStruct((n, 2), jnp.float32),
    )(S, u, dinv2d, W4p, b4.reshape(1, 2))


def kernel(x, edge_index, W1, b1, g1, be1, W2, b2, g2, be2, W3, b3, g3, be3,
           W4, b4):
    n = x.shape[0]
    n_edges = edge_index.shape[1]
    quant = NW * 2560
    epad = quant * -(-n_edges // quant)
    npe = epad - n_edges
    pad_idx = jnp.arange(npe, dtype=jnp.int32)
    src_p = jnp.concatenate([edge_index[0], pad_idx % n])
    dst_p = jnp.concatenate([edge_index[1], n + pad_idx % (NPAD - n)])

    deg_k = _make_deg_kernel(epad)
    scat = _make_scatter_kernel(epad)

    degp = deg_k(dst_p)
    degp_t = degp.T

    dinv2d, u0 = _tc_first(x, degp_t)

    S = scat(src_p, dst_p, u0)
    u1 = _tc_layer(S, u0, dinv2d, W1, b1, g1, be1)

    S = scat(src_p, dst_p, u1)
    u2 = _tc_layer(S, u1, dinv2d, W2, b2, g2, be2)

    S = scat(src_p, dst_p, u2)
    u3 = _tc_layer(S, u2, dinv2d, W3, b3, g3, be3)

    S = scat(src_p, dst_p, u3)
    W4p = jnp.pad(W4, ((0, F - W4.shape[0]), (0, 0)))
    return _tc_final(S, u3, dinv2d, W4p, b4)


# speedup vs baseline: 1.0109x; 1.0109x over previous
import functools

import jax
import jax.numpy as jnp
from jax import lax
from jax.experimental import pallas as pl
from jax.experimental.pallas import tpu as pltpu
from jax.experimental.pallas import tpu_sc as plsc

NC = 2
NS = 16
NW = NC * NS
NPAD = 10240
K = 128
F = 128


def _sc_mesh():
    return plsc.VectorSubcoreMesh(
        core_axis_name="c", subcore_axis_name="s", num_cores=NC, num_subcores=NS
    )


def _make_deg_kernel(n_edges):
    epw = n_edges // NW
    nchunks = epw // K
    assert nchunks % 4 == 0 and epw % K == 0
    ept = NPAD // NS

    @functools.partial(
        pl.kernel,
        out_type=jax.ShapeDtypeStruct((NC, NPAD), jnp.float32),
        mesh=_sc_mesh(),
        scratch_types=[
            [pltpu.VMEM((K,), jnp.int32) for _ in range(4)],
            pltpu.VMEM((K,), jnp.float32),
            pltpu.VMEM((ept,), jnp.float32),
            pltpu.VMEM_SHARED((NPAD,), jnp.float32),
            [pltpu.SemaphoreType.DMA for _ in range(4)],
            [pltpu.SemaphoreType.DMA for _ in range(4)],
        ],
    )
    def deg_kernel(dst_hbm, out_hbm, dstb, ones, zb, acc, semD, semS):
        c = lax.axis_index("c")
        s = lax.axis_index("s")
        w = c * NS + s
        base = w * epw
        for q in range(4):
            pltpu.async_copy(dst_hbm.at[pl.ds(base + q * K, K)], dstb[q], semD[q])
        one16 = jnp.full((16,), 1.0, jnp.float32)
        zero16 = jnp.zeros((16,), jnp.float32)
        for j in range(K // 16):
            ones[pl.ds(j * 16, 16)] = one16

        def zfill(i, carry):
            zb[pl.ds(i * 16, 16)] = zero16
            return carry

        lax.fori_loop(0, ept // 16, zfill, 0)
        pltpu.sync_copy(zb, acc.at[pl.ds(s * ept, ept)])
        plsc.subcore_barrier()

        def body(i, carry):
            c0 = 4 * i
            for q in range(4):
                pltpu.make_async_copy(
                    dst_hbm.at[pl.ds(base + (c0 + q) * K, K)], dstb[q], semD[q]
                ).wait()
                pltpu.async_copy(ones, acc.at[dstb[q]], semS[q], add=True)
            for q in range(4):
                pltpu.make_async_copy(ones, acc.at[dstb[q]], semS[q]).wait()

                @pl.when(c0 + q + 4 < nchunks)
                def _():
                    pltpu.async_copy(
                        dst_hbm.at[pl.ds(base + (c0 + q + 4) * K, K)],
                        dstb[q], semD[q],
                    )

            return carry

        lax.fori_loop(0, nchunks // 4, body, 0)
        plsc.subcore_barrier()
        pltpu.sync_copy(acc.at[pl.ds(s * ept, ept)], zb)
        pltpu.sync_copy(zb, out_hbm.at[c].at[pl.ds(s * ept, ept)])

    return deg_kernel


KS = 80
RD = 4
GD = RD // 2
IR = 2 * RD


def _make_scatter_kernel(n_edges):
    epw = n_edges // NW
    nchunks = epw // KS
    assert nchunks % IR == 0 and epw % KS == 0
    rpt = NPAD // NS
    bounce_rows = 40

    @functools.partial(
        pl.kernel,
        out_type=jax.ShapeDtypeStruct((NC, NPAD, F), jnp.float32),
        mesh=_sc_mesh(),
        scratch_types=[
            [pltpu.VMEM((KS,), jnp.int32) for _ in range(IR)],
            [pltpu.VMEM((KS,), jnp.int32) for _ in range(IR)],
            [pltpu.VMEM((KS, F), jnp.float32) for _ in range(RD)],
            pltpu.VMEM((bounce_rows, F), jnp.float32),
            pltpu.VMEM_SHARED((NPAD, F), jnp.float32),
            [pltpu.SemaphoreType.DMA for _ in range(IR)],
            [pltpu.SemaphoreType.DMA for _ in range(RD)],
            [pltpu.SemaphoreType.DMA for _ in range(RD)],
            pltpu.SemaphoreType.DMA,
        ],
    )
    def scatter_kernel(src_hbm, dst_hbm, u_hbm, out_hbm,
                       srcb, dstb, rows, bounce, acc,
                       semI, semG, semS, semZ):
        c = lax.axis_index("c")
        s = lax.axis_index("s")
        w = c * NS + s
        base = w * epw

        def start_idx(ch, slot):
            pltpu.async_copy(src_hbm.at[pl.ds(base + ch * KS, KS)],
                             srcb[slot], semI[slot])
            pltpu.async_copy(dst_hbm.at[pl.ds(base + ch * KS, KS)],
                             dstb[slot], semI[slot])

        def wait_idx(ch, slot):
            pltpu.make_async_copy(src_hbm.at[pl.ds(base + ch * KS, KS)],
                                  srcb[slot], semI[slot]).wait()
            pltpu.make_async_copy(dst_hbm.at[pl.ds(base + ch * KS, KS)],
                                  dstb[slot], semI[slot]).wait()

        def start_gather(slot8, slot4):
            pltpu.async_copy(u_hbm.at[srcb[slot8]], rows[slot4], semG[slot4])

        def wait_gather(slot8, slot4):
            pltpu.make_async_copy(u_hbm.at[srcb[slot8]], rows[slot4],
                                  semG[slot4]).wait()

        def start_scatter(slot8, slot4):
            pltpu.async_copy(rows[slot4], acc.at[dstb[slot8]], semS[slot4],
                             add=True)

        def wait_scatter(slot8, slot4):
            pltpu.make_async_copy(rows[slot4], acc.at[dstb[slot8]],
                                  semS[slot4]).wait()

        for q in range(IR - GD):
            start_idx(q, q)
        for q in range(GD):
            wait_idx(q, q)
            start_gather(q, q)

        zero16 = jnp.zeros((16,), jnp.float32)

        def zfill(i, carry):
            for j in range(F // 16):
                bounce[i, pl.ds(j * 16, 16)] = zero16
            return carry

        lax.fori_loop(0, bounce_rows, zfill, 0)
        for t in range(rpt // bounce_rows):
            pltpu.async_copy(
                bounce, acc.at[pl.ds(s * rpt + t * bounce_rows, bounce_rows)],
                semZ,
            )
        for t in range(rpt // bounce_rows):
            pltpu.make_async_copy(
                bounce, acc.at[pl.ds(s * rpt + t * bounce_rows, bounce_rows)],
                semZ,
            ).wait()
        plsc.subcore_barrier()

        def body(i, carry):
            c0 = IR * i
            for q in range(IR):
                ch = c0 + q

                @pl.when(ch >= GD)
                def _():
                    wait_scatter((q - GD) % IR, (q - GD) % RD)

                @pl.when(ch + IR - GD < nchunks)
                def _():
                    start_idx(ch + IR - GD, (q - GD) % IR)

                @pl.when(ch + GD < nchunks)
                def _():
                    wait_idx(ch + GD, (q + GD) % IR)
                    start_gather((q + GD) % IR, (q + GD) % RD)

                wait_gather(q, q % RD)
                start_scatter(q, q % RD)
            return carry

        lax.fori_loop(0, nchunks // IR, body, 0)
        for ch in range(nchunks - GD, nchunks):
            wait_scatter(ch % IR, ch % RD)
        plsc.subcore_barrier()
        pltpu.async_copy(acc.at[pl.ds(s * rpt, rpt)],
                         out_hbm.at[c].at[pl.ds(s * rpt, rpt)], semZ)
        pltpu.make_async_copy(acc.at[pl.ds(s * rpt, rpt)],
                              out_hbm.at[c].at[pl.ds(s * rpt, rpt)], semZ).wait()

    return scatter_kernel


def _tc_first(x, degp_t):
    n = x.shape[0]

    def body(x_ref, degp_ref, dinv_ref, u_ref):
        p = degp_ref[...]
        deg = p[:, 0:1] + p[:, 1:2] + 1.0
        dinv = lax.rsqrt(deg)[:n]
        d2 = jnp.broadcast_to(dinv, (n, F))
        dinv_ref[...] = d2
        u_ref[...] = d2 * x_ref[...]

    return pl.pallas_call(
        body,
        out_shape=(
            jax.ShapeDtypeStruct((n, F), jnp.float32),
            jax.ShapeDtypeStruct((n, F), jnp.float32),
        ),
    )(x, degp_t)


def _tc_layer(S, u, dinv2d, W, b, g, be):
    n = u.shape[0]
    fout = W.shape[1]

    def body(s_ref, u_ref, dinv_ref, w_ref, b_ref, g_ref, be_ref, out_ref):
        d2 = dinv_ref[...]
        agg = s_ref[0, :n, :] + s_ref[1, :n, :] + u_ref[...]
        h = jnp.dot(agg, w_ref[...], preferred_element_type=jnp.float32)
        t = d2[:, :fout] * h + b_ref[...]
        t = jnp.maximum(t, 0.0)
        mu = jnp.mean(t, axis=0, keepdims=True)
        var = jnp.mean((t - mu) ** 2, axis=0, keepdims=True)
        t = g_ref[...] * (t - mu) * lax.rsqrt(var + 1e-5) + be_ref[...]
        out_ref[:, :fout] = d2[:, :fout] * t
        if fout < F:
            out_ref[:, fout:] = jnp.zeros((n, F - fout), jnp.float32)

    return pl.pallas_call(
        body,
        out_shape=jax.ShapeDtypeStruct((n, F), jnp.float32),
    )(S, u, dinv2d, W, b.reshape(1, fout), g.reshape(1, fout), be.reshape(1, fout))


def _tc_final(S, u, dinv2d, W4p, b4):
    n = u.shape[0]

    def body(s_ref, u_ref, dinv_ref, w_ref, b_ref, out_ref):
        d2 = dinv_ref[...]
        agg = s_ref[0, :n, :] + s_ref[1, :n, :] + u_ref[...]
        h = jnp.dot(agg, w_ref[...], preferred_element_type=jnp.float32)
        t = d2[:, :2] * h + b_ref[...]
        a = t[:, 0:1]
        bb = t[:, 1:2]
        m = jnp.maximum(a, bb)
        lse = m + jnp.log(jnp.exp(a - m) + jnp.exp(bb - m))
        out_ref[...] = jnp.concatenate([a - lse, bb - lse], axis=1)

    return pl.pallas_call(
        body,
        out_shape=jax.ShapeDtypeStruct((n, 2), jnp.float32),
    )(S, u, dinv2d, W4p, b4.reshape(1, 2))


def kernel(x, edge_index, W1, b1, g1, be1, W2, b2, g2, be2, W3, b3, g3, be3,
           W4, b4):
    n = x.shape[0]
    n_edges = edge_index.shape[1]
    quant = NW * 2560
    epad = quant * -(-n_edges // quant)
    npe = epad - n_edges
    pad_idx = jnp.arange(npe, dtype=jnp.int32)
    src_p = jnp.concatenate([edge_index[0], pad_idx % n])
    dst_p = jnp.concatenate([edge_index[1], n + pad_idx % (NPAD - n)])

    deg_k = _make_deg_kernel(epad)
    scat = _make_scatter_kernel(epad)

    degp = deg_k(dst_p)
    degp_t = degp.T

    dinv2d, u0 = _tc_first(x, degp_t)

    S = scat(src_p, dst_p, u0)
    u1 = _tc_layer(S, u0, dinv2d, W1, b1, g1, be1)

    S = scat(src_p, dst_p, u1)
    u2 = _tc_layer(S, u1, dinv2d, W2, b2, g2, be2)

    S = scat(src_p, dst_p, u2)
    u3 = _tc_layer(S, u2, dinv2d, W3, b3, g3, be3)

    S = scat(src_p, dst_p, u3)
    W4p = jnp.pad(W4, ((0, F - W4.shape[0]), (0, 0)))
    return _tc_final(S, u3, dinv2d, W4p, b4)
